# baseline (speedup 1.0000x reference)
import functools

import jax
import jax.numpy as jnp
from jax import lax
from jax.experimental import pallas as pl
from jax.experimental.pallas import tpu as pltpu
from jax.experimental.pallas import tpu_sc as plsc

N_NODES = 10000
N_EDGES = 320000
D_FEAT = 128
HIDDEN = 256
NEG_SLOPE = 0.2

E_CHUNK = 4000
N_CHUNKS = N_EDGES // E_CHUNK
STEPS = E_CHUNK // 16
UNROLL = 10

NC = 2
NS = 16
NW = NC * NS
F_PER_W = D_FEAT // NW
W_WORDS = F_PER_W * N_NODES


def _full16(v, dtype=jnp.int32):
    return jnp.full((16,), v, dtype=dtype)


def _sc_body(xt_hbm, src_hbm, dst_hbm, perm_hbm, out_hbm,
             a_v, b_v, inv_v, perm_v, se0_v, se1_v, de0_v, de1_v,
             sem_s0, sem_s1, sem_d0, sem_d1):
    se_v = (se0_v, se1_v)
    de_v = (de0_v, de1_v)
    sem_s = (sem_s0, sem_s1)
    sem_d = (sem_d0, sem_d1)
    wid = lax.axis_index("s") * NC + lax.axis_index("c")
    base_w = wid * W_WORDS

    nvec = N_NODES // 16
    zeros16 = _full16(0.0, jnp.float32)
    ones16 = _full16(1.0, jnp.float32)

    @plsc.parallel_loop(0, nvec, unroll=4)
    def _init_deg(i):
        inv_v[pl.ds(i * 16, 16)] = ones16

    def invert_deg():
        @plsc.parallel_loop(0, nvec, unroll=4)
        def _inv_deg(i):
            sl = pl.ds(i * 16, 16)
            inv_v[sl] = ones16 / inv_v[sl]

    def edge_round(count_deg=False):
        def start(chunk, buf):
            sl = pl.ds(chunk * E_CHUNK, E_CHUNK)
            pltpu.async_copy(src_hbm.at[sl], se_v[buf], sem_s[buf])
            pltpu.async_copy(dst_hbm.at[sl], de_v[buf], sem_d[buf])

        def wait(chunk, buf):
            sl = pl.ds(chunk * E_CHUNK, E_CHUNK)
            pltpu.make_async_copy(src_hbm.at[sl], se_v[buf], sem_s[buf]).wait()
            pltpu.make_async_copy(dst_hbm.at[sl], de_v[buf], sem_d[buf]).wait()

        start(0, 0)

        def pair(c2, _):
            for b in range(2):
                chunk = c2 * 2 + b
                wait(chunk, b)

                @pl.when(chunk + 1 < N_CHUNKS)
                def _():
                    start(chunk + 1, 1 - b)

                sbuf = se_v[b]
                dbuf = de_v[b]

                @plsc.parallel_loop(0, STEPS, unroll=UNROLL)
                def _edges(i):
                    sl = pl.ds(i * 16, 16)
                    s16 = sbuf[sl]
                    d16 = dbuf[sl]
                    for f in range(F_PER_W):
                        if f:
                            s16 = s16 + N_NODES
                            d16 = d16 + N_NODES
                        elif count_deg:
                            plsc.addupdate_scatter(inv_v, [d16], ones16)
                        v = plsc.load_gather(a_v, [s16])
                        plsc.addupdate_scatter(b_v, [d16], v)
            return 0
        lax.fori_loop(0, N_CHUNKS // 2, pair, 0)

    def finalize_round():
        @plsc.parallel_loop(0, nvec, unroll=4)
        def _fin(i):
            nsl = pl.ds(i * 16, 16)
            iv = inv_v[nsl]
            for f in range(F_PER_W):
                sl = pl.ds(f * N_NODES + i * 16, 16)
                t = (b_v[sl] + a_v[sl]) * iv
                a_v[sl] = t
                b_v[sl] = zeros16

    def zero_b():
        @plsc.parallel_loop(0, nvec, unroll=4)
        def _z(i):
            for f in range(F_PER_W):
                b_v[pl.ds(f * N_NODES + i * 16, 16)] = zeros16

    for p in range(3):
        if p == 0:
            pltpu.sync_copy(xt_hbm.at[pl.ds(base_w, W_WORDS)], a_v)
            zero_b()
        else:
            pltpu.sync_copy(xt_hbm.at[pl.ds(base_w, W_WORDS)], b_v)
            pltpu.sync_copy(perm_hbm.at[pl.ds((p - 1) * N_NODES, N_NODES)],
                            perm_v)

            @plsc.parallel_loop(0, nvec, unroll=4)
            def _permute(i):
                p16 = perm_v[pl.ds(i * 16, 16)]
                for f in range(F_PER_W):
                    if f:
                        p16 = p16 + N_NODES
                    a_v[pl.ds(f * N_NODES + i * 16, 16)] = (
                        plsc.load_gather(b_v, [p16]))
            zero_b()

        for r in range(2):
            edge_round(count_deg=(p == 0 and r == 0))
            if p == 0 and r == 0:
                invert_deg()
            finalize_round()

        pltpu.sync_copy(a_v, out_hbm.at[pl.ds(p * D_FEAT * N_NODES + base_w,
                                              W_WORDS)])


@functools.cache
def _propagate_sc():
    return pl.kernel(
        _sc_body,
        out_type=jax.ShapeDtypeStruct((3 * D_FEAT * N_NODES,), jnp.float32),
        mesh=plsc.VectorSubcoreMesh(core_axis_name="c", subcore_axis_name="s",
                                    num_cores=NC, num_subcores=NS),
        compiler_params=pltpu.CompilerParams(needs_layout_passes=False),
        scratch_types=[
            pltpu.VMEM((W_WORDS,), jnp.float32),
            pltpu.VMEM((W_WORDS,), jnp.float32),
            pltpu.VMEM((N_NODES,), jnp.float32),
            pltpu.VMEM((N_NODES,), jnp.int32),
            pltpu.VMEM((E_CHUNK,), jnp.int32),
            pltpu.VMEM((E_CHUNK,), jnp.int32),
            pltpu.VMEM((E_CHUNK,), jnp.int32),
            pltpu.VMEM((E_CHUNK,), jnp.int32),
            pltpu.SemaphoreType.DMA,
            pltpu.SemaphoreType.DMA,
            pltpu.SemaphoreType.DMA,
            pltpu.SemaphoreType.DMA,
        ],
    )


def _softplus(z):
    return jnp.maximum(z, 0.0) + jnp.log1p(jnp.exp(-jnp.abs(z)))


def _pool_body(x_ref, batch_ref, w1b_ref, b1_ref, gpart_ref):
    x = x_ref[...]
    batch = batch_ref[...]
    gids = lax.broadcasted_iota(jnp.int32, (1, 64), 1)
    onehot = (batch == gids).astype(jnp.float32)

    cnt = jnp.sum(onehot, axis=0, keepdims=True)
    gsum = lax.dot_general(onehot, x, (((0,), (0,)), ((), ())),
                           preferred_element_type=jnp.float32)
    gmean = gsum / jnp.maximum(cnt, 1.0).T

    neg_inf = jnp.float32(-3.0e38)
    mrows = []
    for g in range(64):
        mask = batch == g
        mg = jnp.max(jnp.where(mask, x, neg_inf), axis=0, keepdims=True)
        mrows.append(mg)
    gmax = jnp.concatenate(mrows, axis=0)

    ge_mean = jnp.maximum(gmean, 0.0)
    ge_max = jnp.maximum(gmax, 0.0)
    w1b = w1b_ref[...]
    gpart_ref[...] = (
        lax.dot_general(ge_mean, w1b[:128], (((1,), (0,)), ((), ())),
                        preferred_element_type=jnp.float32)
        + lax.dot_general(ge_max, w1b[128:], (((1,), (0,)), ((), ())),
                          preferred_element_type=jnp.float32)
        + b1_ref[...])


def _est_body(prop_ref, batch_ref, gpart_ref, w1a_ref,
              w2_ref, b2_ref, a_ref, pred_ref, loss_ref):
    batch = batch_ref[...]
    gids = lax.broadcasted_iota(jnp.int32, (1, 64), 1)
    onehot = (batch == gids).astype(jnp.float32)
    grows = lax.dot_general(onehot, gpart_ref[...], (((1,), (0,)), ((), ())),
                            preferred_element_type=jnp.float32)

    w1a = w1a_ref[...]
    w2 = w2_ref[...]
    b2 = b2_ref[...]
    a = a_ref[...]

    preds = []
    for p in range(3):
        h = prop_ref[p]
        if p == 0:
            act = jnp.maximum(h, 0.0)
        else:
            act = jnp.where(h >= 0, h, a * h)
        z = lax.dot_general(act, w1a, (((0,), (0,)), ((), ())),
                            preferred_element_type=jnp.float32)
        z = z + grows
        z = jnp.where(z >= 0, z, NEG_SLOPE * z)
        preds.append(
            lax.dot_general(z, w2, (((1,), (0,)), ((), ())),
                            preferred_element_type=jnp.float32) + b2)

    pred_ref[...] = preds[0]
    loss = (jnp.sum(_softplus(-preds[0])) / N_NODES
            + (jnp.sum(_softplus(preds[1])) + jnp.sum(_softplus(preds[2])))
            / (2.0 * N_NODES))
    loss_ref[...] = loss.reshape(1, 1)


def _pool_tc(x, batch2, w1b, b1):
    return pl.pallas_call(
        _pool_body,
        out_shape=jax.ShapeDtypeStruct((64, HIDDEN), jnp.float32),
    )(x, batch2, w1b, b1)


def _est_tc(prop, batch2, gpart, w1a, w2, b2, a):
    return pl.pallas_call(
        _est_body,
        out_shape=(
            jax.ShapeDtypeStruct((N_NODES, 1), jnp.float32),
            jax.ShapeDtypeStruct((1, 1), jnp.float32),
        ),
    )(prop, batch2, gpart, w1a, w2, b2, a)


def kernel(x, edge_index, batch, prelu_a, W1, b1, W2, b2):
    n = x.shape[0]
    perm1 = jax.random.permutation(jax.random.key(1), n).astype(jnp.int32)
    perm2 = jax.random.permutation(jax.random.key(2), n).astype(jnp.int32)
    perms = jnp.concatenate([perm1, perm2])

    xt = x.T.reshape(-1)
    src = edge_index[0]
    dst = edge_index[1]

    batch2 = batch.reshape(n, 1)
    gpart = _pool_tc(x, batch2, W1[D_FEAT:], b1.reshape(1, HIDDEN))

    prop = _propagate_sc()(xt, src, dst, perms)
    prop = prop.reshape(3, D_FEAT, N_NODES)

    pred_xy, loss = _est_tc(
        prop, batch2, gpart, W1[:D_FEAT],
        W2, b2.reshape(1, 1), prelu_a.reshape(1, 1))
    return (pred_xy, loss[0, 0])


# speedup vs baseline: 1.0164x; 1.0164x over previous
import functools

import jax
import jax.numpy as jnp
from jax import lax
from jax.experimental import pallas as pl
from jax.experimental.pallas import tpu as pltpu
from jax.experimental.pallas import tpu_sc as plsc

N_NODES = 10000
N_EDGES = 320000
D_FEAT = 128
HIDDEN = 256
NEG_SLOPE = 0.2

E_CHUNK = 4000
N_CHUNKS = N_EDGES // E_CHUNK
STEPS = E_CHUNK // 16
UNROLL = 5

NC = 2
NS = 16
NW = NC * NS
F_PER_W = D_FEAT // NW
W_WORDS = F_PER_W * N_NODES


def _full16(v, dtype=jnp.int32):
    return jnp.full((16,), v, dtype=dtype)


def _sc_body(xt_hbm, src_hbm, dst_hbm, perm_hbm, out_hbm,
             a_v, b_v, inv_v, perm_v, se0_v, se1_v, de0_v, de1_v,
             sem_s0, sem_s1, sem_d0, sem_d1):
    se_v = (se0_v, se1_v)
    de_v = (de0_v, de1_v)
    sem_s = (sem_s0, sem_s1)
    sem_d = (sem_d0, sem_d1)
    wid = lax.axis_index("s") * NC + lax.axis_index("c")
    base_w = wid * W_WORDS

    nvec = N_NODES // 16
    zeros16 = _full16(0.0, jnp.float32)
    ones16 = _full16(1.0, jnp.float32)

    @plsc.parallel_loop(0, nvec, unroll=4)
    def _init_deg(i):
        inv_v[pl.ds(i * 16, 16)] = ones16

    def invert_deg():
        @plsc.parallel_loop(0, nvec, unroll=4)
        def _inv_deg(i):
            sl = pl.ds(i * 16, 16)
            inv_v[sl] = ones16 / inv_v[sl]

    def edge_round(count_deg=False):
        def start(chunk, buf):
            sl = pl.ds(chunk * E_CHUNK, E_CHUNK)
            pltpu.async_copy(src_hbm.at[sl], se_v[buf], sem_s[buf])
            pltpu.async_copy(dst_hbm.at[sl], de_v[buf], sem_d[buf])

        def wait(chunk, buf):
            sl = pl.ds(chunk * E_CHUNK, E_CHUNK)
            pltpu.make_async_copy(src_hbm.at[sl], se_v[buf], sem_s[buf]).wait()
            pltpu.make_async_copy(dst_hbm.at[sl], de_v[buf], sem_d[buf]).wait()

        start(0, 0)

        def pair(c2, _):
            for b in range(2):
                chunk = c2 * 2 + b
                wait(chunk, b)

                @pl.when(chunk + 1 < N_CHUNKS)
                def _():
                    start(chunk + 1, 1 - b)

                sbuf = se_v[b]
                dbuf = de_v[b]

                @plsc.parallel_loop(0, STEPS, unroll=UNROLL)
                def _edges(i):
                    sl = pl.ds(i * 16, 16)
                    s16 = sbuf[sl]
                    d16 = dbuf[sl]
                    for f in range(F_PER_W):
                        if f:
                            s16 = s16 + N_NODES
                            d16 = d16 + N_NODES
                        elif count_deg:
                            plsc.addupdate_scatter(inv_v, [d16], ones16)
                        v = plsc.load_gather(a_v, [s16])
                        plsc.addupdate_scatter(b_v, [d16], v)
            return 0
        lax.fori_loop(0, N_CHUNKS // 2, pair, 0)

    def finalize_round():
        @plsc.parallel_loop(0, nvec, unroll=4)
        def _fin(i):
            nsl = pl.ds(i * 16, 16)
            iv = inv_v[nsl]
            for f in range(F_PER_W):
                sl = pl.ds(f * N_NODES + i * 16, 16)
                t = (b_v[sl] + a_v[sl]) * iv
                a_v[sl] = t
                b_v[sl] = zeros16

    def zero_b():
        @plsc.parallel_loop(0, nvec, unroll=4)
        def _z(i):
            for f in range(F_PER_W):
                b_v[pl.ds(f * N_NODES + i * 16, 16)] = zeros16

    for p in range(3):
        if p == 0:
            pltpu.sync_copy(xt_hbm.at[pl.ds(base_w, W_WORDS)], a_v)
            zero_b()
        else:
            pltpu.sync_copy(xt_hbm.at[pl.ds(base_w, W_WORDS)], b_v)
            pltpu.sync_copy(perm_hbm.at[pl.ds((p - 1) * N_NODES, N_NODES)],
                            perm_v)

            @plsc.parallel_loop(0, nvec, unroll=4)
            def _permute(i):
                p16 = perm_v[pl.ds(i * 16, 16)]
                for f in range(F_PER_W):
                    if f:
                        p16 = p16 + N_NODES
                    a_v[pl.ds(f * N_NODES + i * 16, 16)] = (
                        plsc.load_gather(b_v, [p16]))
            zero_b()

        for r in range(2):
            edge_round(count_deg=(p == 0 and r == 0))
            if p == 0 and r == 0:
                invert_deg()
            finalize_round()

        pltpu.sync_copy(a_v, out_hbm.at[pl.ds(p * D_FEAT * N_NODES + base_w,
                                              W_WORDS)])


@functools.cache
def _propagate_sc():
    return pl.kernel(
        _sc_body,
        out_type=jax.ShapeDtypeStruct((3 * D_FEAT * N_NODES,), jnp.float32),
        mesh=plsc.VectorSubcoreMesh(core_axis_name="c", subcore_axis_name="s",
                                    num_cores=NC, num_subcores=NS),
        compiler_params=pltpu.CompilerParams(needs_layout_passes=False),
        scratch_types=[
            pltpu.VMEM((W_WORDS,), jnp.float32),
            pltpu.VMEM((W_WORDS,), jnp.float32),
            pltpu.VMEM((N_NODES,), jnp.float32),
            pltpu.VMEM((N_NODES,), jnp.int32),
            pltpu.VMEM((E_CHUNK,), jnp.int32),
            pltpu.VMEM((E_CHUNK,), jnp.int32),
            pltpu.VMEM((E_CHUNK,), jnp.int32),
            pltpu.VMEM((E_CHUNK,), jnp.int32),
            pltpu.SemaphoreType.DMA,
            pltpu.SemaphoreType.DMA,
            pltpu.SemaphoreType.DMA,
            pltpu.SemaphoreType.DMA,
        ],
    )


def _softplus(z):
    return jnp.maximum(z, 0.0) + jnp.log1p(jnp.exp(-jnp.abs(z)))


def _pool_body(x_ref, batch_ref, w1b_ref, b1_ref, gpart_ref):
    x = x_ref[...]
    batch = batch_ref[...]
    gids = lax.broadcasted_iota(jnp.int32, (1, 64), 1)
    onehot = (batch == gids).astype(jnp.float32)

    cnt = jnp.sum(onehot, axis=0, keepdims=True)
    gsum = lax.dot_general(onehot, x, (((0,), (0,)), ((), ())),
                           preferred_element_type=jnp.float32)
    gmean = gsum / jnp.maximum(cnt, 1.0).T

    neg_inf = jnp.float32(-3.0e38)
    mrows = []
    for g in range(64):
        mask = batch == g
        mg = jnp.max(jnp.where(mask, x, neg_inf), axis=0, keepdims=True)
        mrows.append(mg)
    gmax = jnp.concatenate(mrows, axis=0)

    ge_mean = jnp.maximum(gmean, 0.0)
    ge_max = jnp.maximum(gmax, 0.0)
    w1b = w1b_ref[...]
    gpart_ref[...] = (
        lax.dot_general(ge_mean, w1b[:128], (((1,), (0,)), ((), ())),
                        preferred_element_type=jnp.float32)
        + lax.dot_general(ge_max, w1b[128:], (((1,), (0,)), ((), ())),
                          preferred_element_type=jnp.float32)
        + b1_ref[...])


def _est_body(prop_ref, batch_ref, gpart_ref, w1a_ref,
              w2_ref, b2_ref, a_ref, pred_ref, loss_ref):
    batch = batch_ref[...]
    gids = lax.broadcasted_iota(jnp.int32, (1, 64), 1)
    onehot = (batch == gids).astype(jnp.float32)
    grows = lax.dot_general(onehot, gpart_ref[...], (((1,), (0,)), ((), ())),
                            preferred_element_type=jnp.float32)

    w1a = w1a_ref[...]
    w2 = w2_ref[...]
    b2 = b2_ref[...]
    a = a_ref[...]

    preds = []
    for p in range(3):
        h = prop_ref[p]
        if p == 0:
            act = jnp.maximum(h, 0.0)
        else:
            act = jnp.where(h >= 0, h, a * h)
        z = lax.dot_general(act, w1a, (((0,), (0,)), ((), ())),
                            preferred_element_type=jnp.float32)
        z = z + grows
        z = jnp.where(z >= 0, z, NEG_SLOPE * z)
        preds.append(
            lax.dot_general(z, w2, (((1,), (0,)), ((), ())),
                            preferred_element_type=jnp.float32) + b2)

    pred_ref[...] = preds[0]
    loss = (jnp.sum(_softplus(-preds[0])) / N_NODES
            + (jnp.sum(_softplus(preds[1])) + jnp.sum(_softplus(preds[2])))
            / (2.0 * N_NODES))
    loss_ref[...] = loss.reshape(1, 1)


def _pool_tc(x, batch2, w1b, b1):
    return pl.pallas_call(
        _pool_body,
        out_shape=jax.ShapeDtypeStruct((64, HIDDEN), jnp.float32),
    )(x, batch2, w1b, b1)


def _est_tc(prop, batch2, gpart, w1a, w2, b2, a):
    return pl.pallas_call(
        _est_body,
        out_shape=(
            jax.ShapeDtypeStruct((N_NODES, 1), jnp.float32),
            jax.ShapeDtypeStruct((1, 1), jnp.float32),
        ),
    )(prop, batch2, gpart, w1a, w2, b2, a)


def kernel(x, edge_index, batch, prelu_a, W1, b1, W2, b2):
    n = x.shape[0]
    perm1 = jax.random.permutation(jax.random.key(1), n).astype(jnp.int32)
    perm2 = jax.random.permutation(jax.random.key(2), n).astype(jnp.int32)
    perms = jnp.concatenate([perm1, perm2])

    xt = x.T.reshape(-1)
    src = edge_index[0]
    dst = edge_index[1]

    batch2 = batch.reshape(n, 1)
    gpart = _pool_tc(x, batch2, W1[D_FEAT:], b1.reshape(1, HIDDEN))

    prop = _propagate_sc()(xt, src, dst, perms)
    prop = prop.reshape(3, D_FEAT, N_NODES)

    pred_xy, loss = _est_tc(
        prop, batch2, gpart, W1[:D_FEAT],
        W2, b2.reshape(1, 1), prelu_a.reshape(1, 1))
    return (pred_xy, loss[0, 0])
